# baseline (speedup 1.0000x reference)
import functools

import jax
import jax.numpy as jnp
from jax import lax
from jax.experimental import pallas as pl
from jax.experimental.pallas import tpu as pltpu
from jax.experimental.pallas import tpu_sc as plsc

N_NODES = 10000
N_EDGES = 320000
HIDDEN = 128
NUM_RBF = 16
CUTOFF = 1.0
FF = 512

NC, NS, L = 2, 16, 16
NW = NC * NS
EPW = N_EDGES // NW
EPC = N_EDGES // NS
K = 80
NCHUNK = EPC // K
GCHUNK = EPW // K
TW = 3 * HIDDEN


def _pre_body(h_ref, wr_ref, wc_ref, xp_ref, bcat_ref, trow_ref, tcol_ref):
    h = h_ref[...]
    xp = xp_ref[...]
    trow_ref[:, 0:256] = (
        jnp.dot(h, wr_ref[...], preferred_element_type=jnp.float32) + bcat_ref[...])
    trow_ref[:, 256:384] = xp
    tcol_ref[:, 0:256] = jnp.dot(h, wc_ref[...], preferred_element_type=jnp.float32)
    tcol_ref[:, 256:384] = xp


def _precompute(h, wr, wc, xp, bcat):
    nb = 2000
    grid = N_NODES // nb
    return pl.pallas_call(
        _pre_body,
        grid=(grid,),
        in_specs=[
            pl.BlockSpec((nb, HIDDEN), lambda i: (i, 0)),
            pl.BlockSpec((HIDDEN, 256), lambda i: (0, 0)),
            pl.BlockSpec((HIDDEN, 256), lambda i: (0, 0)),
            pl.BlockSpec((nb, HIDDEN), lambda i: (i, 0)),
            pl.BlockSpec((1, 256), lambda i: (0, 0)),
        ],
        out_specs=[
            pl.BlockSpec((nb, TW), lambda i: (i, 0)),
            pl.BlockSpec((nb, TW), lambda i: (i, 0)),
        ],
        out_shape=[
            jax.ShapeDtypeStruct((N_NODES, TW), jnp.float32),
            jax.ShapeDtypeStruct((N_NODES, TW), jnp.float32),
        ],
    )(h, wr, wc, xp, bcat)


GSUP = 25
GNSUP = GCHUNK // GSUP


def _gather_body(trow_hbm, tcol_hbm, row4_hbm, col4_hbm,
                 eadd_hbm,
                 idxr2, idxc2,
                 bufr0, bufc0, bufr1, bufc1,
                 semr0, semc0, semr1, semc1):
    wid = lax.axis_index("c") * NS + lax.axis_index("s")
    base = wid * EPW

    bufrs = (bufr0, bufr1)
    bufcs = (bufc0, bufc1)
    semrs = (semr0, semr1)
    semcs = (semc0, semc1)

    def super_body(u, carry):
        pltpu.sync_copy(row4_hbm.at[wid, u], idxr2)
        pltpu.sync_copy(col4_hbm.at[wid, u], idxc2)
        cr = [None, None]
        cc = [None, None]
        cr[0] = pltpu.async_copy(trow_hbm.at[idxr2.at[0]], bufr0, semr0)
        cc[0] = pltpu.async_copy(tcol_hbm.at[idxc2.at[0]], bufc0, semc0)
        for j in range(GSUP):
            pj = j % 2
            if j + 1 < GSUP:
                cr[1 - pj] = pltpu.async_copy(
                    trow_hbm.at[idxr2.at[j + 1]], bufrs[1 - pj], semrs[1 - pj])
                cc[1 - pj] = pltpu.async_copy(
                    tcol_hbm.at[idxc2.at[j + 1]], bufcs[1 - pj], semcs[1 - pj])
            cr[pj].wait()
            cc[pj].wait()
            br = bufrs[pj]
            bc = bufcs[pj]

            def fuse(e, fcarry, _br=br, _bc=bc):
                for jj in range(16):
                    sl = pl.ds(jj * L, L)
                    _br[e, sl] = _br[e, sl] + _bc[e, sl]
                sl = pl.ds(256, L)
                _br[e, sl] = _br[e, sl] - _bc[e, sl]
                return fcarry

            lax.fori_loop(0, K, fuse, 0)
            pltpu.sync_copy(br, eadd_hbm.at[pl.ds(base + (u * GSUP + j) * K, K)])
        return carry

    lax.fori_loop(0, GNSUP, super_body, 0)


def _gather(trow, tcol, row4, col4):
    mesh = plsc.VectorSubcoreMesh(core_axis_name="c", subcore_axis_name="s")
    f = pl.kernel(
        _gather_body,
        out_type=jax.ShapeDtypeStruct((N_EDGES, TW), jnp.float32),
        mesh=mesh,
        scratch_types=[
            pltpu.VMEM((GSUP, K), jnp.int32),
            pltpu.VMEM((GSUP, K), jnp.int32),
            pltpu.VMEM((K, TW), jnp.float32),
            pltpu.VMEM((K, TW), jnp.float32),
            pltpu.VMEM((K, TW), jnp.float32),
            pltpu.VMEM((K, TW), jnp.float32),
            pltpu.SemaphoreType.DMA,
            pltpu.SemaphoreType.DMA,
            pltpu.SemaphoreType.DMA,
            pltpu.SemaphoreType.DMA,
        ],
    )
    return f(trow, tcol, row4, col4)


def _edge_body(ea_ref, wrbf_ref, cen_ref, w2a_ref, w2c_ref, misc_ref,
               outw_ref, outa_ref):
    ea = ea_ref[...]
    rel = ea[:, 256:259]
    d2 = jnp.sum(rel * rel, axis=1, keepdims=True)
    dist = jnp.sqrt(d2 + 1e-12)
    gamma = 10.0 / (CUTOFF + 1e-6)
    d = dist - cen_ref[...]
    rbf = jnp.exp(-gamma * d * d)
    rc = jnp.dot(rbf, wrbf_ref[...], preferred_element_type=jnp.float32)
    att_h = jax.nn.relu(ea[:, 0:128] + rc[:, 0:128])
    logit = jnp.sum(att_h * w2a_ref[...], axis=1, keepdims=True) + misc_ref[0, 0]
    w = jax.nn.sigmoid(logit)
    crd_h = jax.nn.relu(ea[:, 128:256] + rc[:, 128:256])
    gl = jnp.sum(crd_h * w2c_ref[...], axis=1, keepdims=True) + misc_ref[0, 1]
    gate = jax.nn.sigmoid(gl)
    wg = w * gate
    n = w.shape[0]
    outw_ref[...] = jnp.broadcast_to(w, (n, 16))
    outa_ref[...] = jnp.concatenate(
        [wg * rel, jnp.ones((n, 1), jnp.float32),
         jnp.zeros((n, 12), jnp.float32)], axis=1)


def _edgewise(eadd, wrbf, cen, w2a, w2c, misc):
    be = 8000
    grid = eadd.shape[0] // be
    whole = lambda s: pl.BlockSpec(s, lambda i: tuple(0 for _ in s))
    return pl.pallas_call(
        _edge_body,
        grid=(grid,),
        in_specs=[
            pl.BlockSpec((be, TW), lambda i: (i, 0)),
            whole((NUM_RBF, 256)),
            whole((1, NUM_RBF)),
            whole((1, HIDDEN)),
            whole((1, HIDDEN)),
            whole((1, HIDDEN)),
        ],
        out_specs=[
            pl.BlockSpec((be, 16), lambda i: (i, 0)),
            pl.BlockSpec((be, 16), lambda i: (i, 0)),
        ],
        out_shape=[
            jax.ShapeDtypeStruct((eadd.shape[0], 16), jnp.float32),
            jax.ShapeDtypeStruct((eadd.shape[0], 16), jnp.float32),
        ],
    )(eadd, wrbf, cen, w2a, w2c, misc)


SUP = 10
NSUP = NCHUNK // SUP


def _scatter_body(h_hbm, row2_hbm, col2_hbm, wm_hbm, wa_hbm, z_hbm, out_hbm,
                  idxr2, idxc2, hbuf0, hbuf1, wf0, wf1, msg0, msg1, acc,
                  semh0, semh1, semw0, semw1, sems0, sems1):
    c = lax.axis_index("c")
    s = lax.axis_index("s")

    @pl.when(s == 0)
    def _init():
        pltpu.sync_copy(z_hbm, acc)

    plsc.subcore_barrier()

    hbufs = (hbuf0, hbuf1)
    wfs = (wf0, wf1)
    msgs = (msg0, msg1)
    semhs = (semh0, semh1)
    semws = (semw0, semw1)
    semss = (sems0, sems1)

    @pl.when(c == 0)
    def _msg_flow():
        def super_body(u, carry):
            cr = s * NCHUNK + u * SUP
            pltpu.sync_copy(row2_hbm.at[s, u], idxr2)
            pltpu.sync_copy(col2_hbm.at[s, u], idxc2)
            ch = [None, None]
            cw = [None, None]
            sa = [None, None]
            ch[0] = pltpu.async_copy(h_hbm.at[idxc2.at[0]], hbuf0, semh0)
            cw[0] = pltpu.async_copy(
                wm_hbm.at[pl.ds(cr * K * 16, K * 16)], wf0, semw0)
            for j in range(SUP):
                pj = j % 2
                if j + 1 < SUP:
                    ch[1 - pj] = pltpu.async_copy(
                        h_hbm.at[idxc2.at[j + 1]], hbufs[1 - pj], semhs[1 - pj])
                    cw[1 - pj] = pltpu.async_copy(
                        wm_hbm.at[pl.ds((cr + j + 1) * K * 16, K * 16)],
                        wfs[1 - pj], semws[1 - pj])
                ch[pj].wait()
                cw[pj].wait()
                if j >= 2:
                    sa[pj].wait()
                mb = msgs[pj]
                hb = hbufs[pj]
                wb = wfs[pj]

                def edge(e, ecarry, _mb=mb, _hb=hb, _wb=wb):
                    wv = _wb[pl.ds(e * L, L)]
                    for jj in range(HIDDEN // L):
                        _mb[e, pl.ds(jj * L, L)] = wv * _hb[e, pl.ds(jj * L, L)]
                    return ecarry

                lax.fori_loop(0, K, edge, 0)
                sa[pj] = pltpu.async_copy(mb, acc.at[idxr2.at[j]], semss[pj],
                                          add=True)
            sa[0].wait()
            sa[1].wait()
            return carry

        lax.fori_loop(0, NSUP, super_body, 0)

    @pl.when(c == 1)
    def _aux_flow():
        zv = jnp.zeros((L,), jnp.float32)

        def zrow(e, carry):
            for j in range(1, HIDDEN // L):
                msg0[e, pl.ds(j * L, L)] = zv
                msg1[e, pl.ds(j * L, L)] = zv
            return carry

        lax.fori_loop(0, K, zrow, 0)

        def super_body(u, carry):
            cr = s * NCHUNK + u * SUP
            pltpu.sync_copy(row2_hbm.at[s, u], idxr2)
            cw = [None, None]
            sa = [None, None]
            cw[0] = pltpu.async_copy(
                wa_hbm.at[pl.ds(cr * K * 16, K * 16)], wf0, semw0)
            for j in range(SUP):
                pj = j % 2
                if j + 1 < SUP:
                    cw[1 - pj] = pltpu.async_copy(
                        wa_hbm.at[pl.ds((cr + j + 1) * K * 16, K * 16)],
                        wfs[1 - pj], semws[1 - pj])
                cw[pj].wait()
                if j >= 2:
                    sa[pj].wait()
                mb = msgs[pj]
                wb = wfs[pj]

                def edge(e, ecarry, _mb=mb, _wb=wb):
                    _mb[e, pl.ds(0, L)] = _wb[pl.ds(e * L, L)]
                    return ecarry

                lax.fori_loop(0, K, edge, 0)
                sa[pj] = pltpu.async_copy(mb, acc.at[idxr2.at[j]], semss[pj],
                                          add=True)
            sa[0].wait()
            sa[1].wait()
            return carry

        lax.fori_loop(0, NSUP, super_body, 0)

    plsc.subcore_barrier()

    @pl.when(s == 0)
    def _drain():
        pltpu.sync_copy(acc, out_hbm.at[c])


def _scatter(h, row2, col2, wm, wa, zeros_acc):
    mesh = plsc.VectorSubcoreMesh(core_axis_name="c", subcore_axis_name="s")
    f = pl.kernel(
        _scatter_body,
        out_type=jax.ShapeDtypeStruct((NC, N_NODES, HIDDEN), jnp.float32),
        mesh=mesh,
        scratch_types=[
            pltpu.VMEM((SUP, K), jnp.int32),
            pltpu.VMEM((SUP, K), jnp.int32),
            pltpu.VMEM((K, HIDDEN), jnp.float32),
            pltpu.VMEM((K, HIDDEN), jnp.float32),
            pltpu.VMEM((K * 16,), jnp.float32),
            pltpu.VMEM((K * 16,), jnp.float32),
            pltpu.VMEM((K, HIDDEN), jnp.float32),
            pltpu.VMEM((K, HIDDEN), jnp.float32),
            pltpu.VMEM_SHARED((N_NODES, HIDDEN), jnp.float32),
            pltpu.SemaphoreType.DMA,
            pltpu.SemaphoreType.DMA,
            pltpu.SemaphoreType.DMA,
            pltpu.SemaphoreType.DMA,
            pltpu.SemaphoreType.DMA,
            pltpu.SemaphoreType.DMA,
        ],
    )
    return f(h, row2, col2, wm, wa, zeros_acc)


def _ln(v, g, b):
    mu = jnp.mean(v, axis=-1, keepdims=True)
    var = jnp.mean((v - mu) ** 2, axis=-1, keepdims=True)
    return (v - mu) / jnp.sqrt(var + 1e-5) * g + b


def _final_body(h_ref, x_ref, acc_ref, mw1h_ref, mw1a_ref, mb1_ref,
                mw2_ref, mb2_ref, fw1_ref, fb1_ref, fw2_ref, fb2_ref,
                g1_ref, b1_ref, g2_ref, b2_ref, xn_ref, h2_ref):
    aux = acc_ref[1]
    cnt = jnp.maximum(aux[:, 3:4], 1.0)
    agg = acc_ref[0] / cnt
    dx = aux[:, 0:3] / cnt
    xn_ref[...] = x_ref[...] + dx
    h = h_ref[...]
    t = jax.nn.relu(
        jnp.dot(h, mw1h_ref[...], preferred_element_type=jnp.float32)
        + jnp.dot(agg, mw1a_ref[...], preferred_element_type=jnp.float32)
        + mb1_ref[...])
    delta_h = jnp.dot(t, mw2_ref[...], preferred_element_type=jnp.float32) + mb2_ref[...]
    h1 = _ln(h + delta_h, g1_ref[...], b1_ref[...])
    h1 = jax.nn.relu(h1)
    f = jax.nn.relu(jnp.dot(h1, fw1_ref[...], preferred_element_type=jnp.float32) + fb1_ref[...])
    ffn = jnp.dot(f, fw2_ref[...], preferred_element_type=jnp.float32) + fb2_ref[...]
    h2_ref[...] = _ln(h1 + ffn, g2_ref[...], b2_ref[...])


def _finalize(h, x, acc, p):
    nb = 5000
    grid = N_NODES // nb
    whole = lambda s: pl.BlockSpec(s, lambda i: tuple(0 for _ in s))
    return pl.pallas_call(
        _final_body,
        grid=(grid,),
        in_specs=[
            pl.BlockSpec((nb, HIDDEN), lambda i: (i, 0)),
            pl.BlockSpec((nb, 3), lambda i: (i, 0)),
            pl.BlockSpec((NC, nb, HIDDEN), lambda i: (0, i, 0)),
            whole((HIDDEN, HIDDEN)),
            whole((HIDDEN, HIDDEN)),
            whole((1, HIDDEN)),
            whole((HIDDEN, HIDDEN)),
            whole((1, HIDDEN)),
            whole((HIDDEN, FF)),
            whole((1, FF)),
            whole((FF, HIDDEN)),
            whole((1, HIDDEN)),
            whole((1, HIDDEN)),
            whole((1, HIDDEN)),
            whole((1, HIDDEN)),
            whole((1, HIDDEN)),
        ],
        out_specs=[
            pl.BlockSpec((nb, 3), lambda i: (i, 0)),
            pl.BlockSpec((nb, HIDDEN), lambda i: (i, 0)),
        ],
        out_shape=[
            jax.ShapeDtypeStruct((N_NODES, 3), jnp.float32),
            jax.ShapeDtypeStruct((N_NODES, HIDDEN), jnp.float32),
        ],
    )(h, x, acc,
      p['msg_w1'][:HIDDEN], p['msg_w1'][HIDDEN:], p['msg_b1'].reshape(1, -1),
      p['msg_w2'], p['msg_b2'].reshape(1, -1),
      p['ffn_w1'], p['ffn_b1'].reshape(1, -1),
      p['ffn_w2'], p['ffn_b2'].reshape(1, -1),
      p['ln1_g'].reshape(1, -1), p['ln1_b'].reshape(1, -1),
      p['ln2_g'].reshape(1, -1), p['ln2_b'].reshape(1, -1))


def kernel(x, h, edge_index, params):
    p = params
    row = edge_index[0]
    col = edge_index[1]
    xp = jnp.pad(x, ((0, 0), (0, HIDDEN - 3)))
    wr = jnp.concatenate([p['att_w1'][:HIDDEN], p['coord_w1'][:HIDDEN]], axis=1)
    wc = jnp.concatenate([p['att_w1'][HIDDEN:2 * HIDDEN],
                          p['coord_w1'][HIDDEN:2 * HIDDEN]], axis=1)
    wrbf = jnp.concatenate([p['att_w1'][2 * HIDDEN:],
                            p['coord_w1'][2 * HIDDEN:]], axis=1)
    cen = jnp.linspace(0.0, CUTOFF, NUM_RBF).reshape(1, -1).astype(jnp.float32)
    bcat = jnp.concatenate([p['att_b1'], p['coord_b1']]).reshape(1, -1)
    w2a = p['att_w2'].reshape(1, -1)
    w2c = p['coord_w2'].reshape(1, -1)
    misc = jnp.zeros((1, HIDDEN), jnp.float32)
    misc = misc.at[0, 0].set(p['att_b2'][0]).at[0, 1].set(p['coord_b2'][0])

    trow, tcol = _precompute(h, wr, wc, xp, bcat)
    eadd = _gather(trow, tcol, row.reshape(NW, GNSUP, GSUP, K),
                   col.reshape(NW, GNSUP, GSUP, K))
    wm, wa = _edgewise(eadd, wrbf, cen, w2a, w2c, misc)
    zeros_acc = jnp.zeros((N_NODES, HIDDEN), jnp.float32)
    acc = _scatter(h, row.reshape(NS, NSUP, SUP, K), col.reshape(NS, NSUP, SUP, K),
                   wm.reshape(-1), wa.reshape(-1), zeros_acc)
    x_new, h2 = _finalize(h, x, acc, p)
    return (x_new, h2)


# speedup vs baseline: 1.0752x; 1.0752x over previous
import functools

import jax
import jax.numpy as jnp
from jax import lax
from jax.experimental import pallas as pl
from jax.experimental.pallas import tpu as pltpu
from jax.experimental.pallas import tpu_sc as plsc

N_NODES = 10000
N_EDGES = 320000
HIDDEN = 128
NUM_RBF = 16
CUTOFF = 1.0
FF = 512

NC, NS, L = 2, 16, 16
NW = NC * NS
EPW = N_EDGES // NW
EPC = N_EDGES // NS
K = 80
NCHUNK = EPC // K
GCHUNK = EPW // K
TW = 3 * HIDDEN


def _pre_body(h_ref, wr_ref, wc_ref, xp_ref, bcat_ref, trow_ref, tcol_ref):
    h = h_ref[...]
    xp = xp_ref[...]
    trow_ref[:, 0:256] = (
        jnp.dot(h, wr_ref[...], preferred_element_type=jnp.float32) + bcat_ref[...])
    trow_ref[:, 256:384] = xp
    tcol_ref[:, 0:256] = jnp.dot(h, wc_ref[...], preferred_element_type=jnp.float32)
    tcol_ref[:, 256:384] = xp


def _precompute(h, wr, wc, xp, bcat):
    nb = 2000
    grid = N_NODES // nb
    return pl.pallas_call(
        _pre_body,
        grid=(grid,),
        in_specs=[
            pl.BlockSpec((nb, HIDDEN), lambda i: (i, 0)),
            pl.BlockSpec((HIDDEN, 256), lambda i: (0, 0)),
            pl.BlockSpec((HIDDEN, 256), lambda i: (0, 0)),
            pl.BlockSpec((nb, HIDDEN), lambda i: (i, 0)),
            pl.BlockSpec((1, 256), lambda i: (0, 0)),
        ],
        out_specs=[
            pl.BlockSpec((nb, TW), lambda i: (i, 0)),
            pl.BlockSpec((nb, TW), lambda i: (i, 0)),
        ],
        out_shape=[
            jax.ShapeDtypeStruct((N_NODES, TW), jnp.float32),
            jax.ShapeDtypeStruct((N_NODES, TW), jnp.float32),
        ],
    )(h, wr, wc, xp, bcat)


GSUP = 25
GNSUP = GCHUNK // GSUP


def _gather_body(trow_hbm, tcol_hbm, row4_hbm, col4_hbm,
                 eadd_hbm,
                 idxr2, idxc2,
                 bufr0, bufc0, bufr1, bufc1,
                 semr0, semc0, semr1, semc1):
    wid = lax.axis_index("c") * NS + lax.axis_index("s")
    base = wid * EPW

    bufrs = (bufr0, bufr1)
    bufcs = (bufc0, bufc1)
    semrs = (semr0, semr1)
    semcs = (semc0, semc1)

    def super_body(u, carry):
        pltpu.sync_copy(row4_hbm.at[wid, u], idxr2)
        pltpu.sync_copy(col4_hbm.at[wid, u], idxc2)
        cr = [None, None]
        cc = [None, None]
        cr[0] = pltpu.async_copy(trow_hbm.at[idxr2.at[0]], bufr0, semr0)
        cc[0] = pltpu.async_copy(tcol_hbm.at[idxc2.at[0]], bufc0, semc0)
        for j in range(GSUP):
            pj = j % 2
            if j + 1 < GSUP:
                cr[1 - pj] = pltpu.async_copy(
                    trow_hbm.at[idxr2.at[j + 1]], bufrs[1 - pj], semrs[1 - pj])
                cc[1 - pj] = pltpu.async_copy(
                    tcol_hbm.at[idxc2.at[j + 1]], bufcs[1 - pj], semcs[1 - pj])
            cr[pj].wait()
            cc[pj].wait()
            br = bufrs[pj]
            bc = bufcs[pj]

            def fuse(e, fcarry, _br=br, _bc=bc):
                for jj in range(16):
                    sl = pl.ds(jj * L, L)
                    _br[e, sl] = _br[e, sl] + _bc[e, sl]
                sl = pl.ds(256, L)
                _br[e, sl] = _br[e, sl] - _bc[e, sl]
                return fcarry

            lax.fori_loop(0, K, fuse, 0)
            pltpu.sync_copy(br, eadd_hbm.at[pl.ds(base + (u * GSUP + j) * K, K)])
        return carry

    lax.fori_loop(0, GNSUP, super_body, 0)


def _gather(trow, tcol, row4, col4):
    mesh = plsc.VectorSubcoreMesh(core_axis_name="c", subcore_axis_name="s")
    f = pl.kernel(
        _gather_body,
        out_type=jax.ShapeDtypeStruct((N_EDGES, TW), jnp.float32),
        mesh=mesh,
        scratch_types=[
            pltpu.VMEM((GSUP, K), jnp.int32),
            pltpu.VMEM((GSUP, K), jnp.int32),
            pltpu.VMEM((K, TW), jnp.float32),
            pltpu.VMEM((K, TW), jnp.float32),
            pltpu.VMEM((K, TW), jnp.float32),
            pltpu.VMEM((K, TW), jnp.float32),
            pltpu.SemaphoreType.DMA,
            pltpu.SemaphoreType.DMA,
            pltpu.SemaphoreType.DMA,
            pltpu.SemaphoreType.DMA,
        ],
    )
    return f(trow, tcol, row4, col4)


def _edge_body(ea_ref, wrbf_ref, cen_ref, w2a_ref, w2c_ref, misc_ref, out_ref):
    ea = ea_ref[...]
    rel = ea[:, 256:259]
    d2 = jnp.sum(rel * rel, axis=1, keepdims=True)
    dist = jnp.sqrt(d2 + 1e-12)
    gamma = 10.0 / (CUTOFF + 1e-6)
    d = dist - cen_ref[...]
    rbf = jnp.exp(-gamma * d * d)
    rc = jnp.dot(rbf, wrbf_ref[...], preferred_element_type=jnp.float32)
    att_h = jax.nn.relu(ea[:, 0:128] + rc[:, 0:128])
    logit = jnp.sum(att_h * w2a_ref[...], axis=1, keepdims=True) + misc_ref[0, 0]
    w = jax.nn.sigmoid(logit)
    crd_h = jax.nn.relu(ea[:, 128:256] + rc[:, 128:256])
    gl = jnp.sum(crd_h * w2c_ref[...], axis=1, keepdims=True) + misc_ref[0, 1]
    gate = jax.nn.sigmoid(gl)
    wg = w * gate
    n = w.shape[0]
    out_ref[...] = jnp.concatenate(
        [jnp.broadcast_to(w, (n, 16)), wg * rel, jnp.ones((n, 1), jnp.float32),
         jnp.zeros((n, 12), jnp.float32)], axis=1)


def _edgewise(eadd, wrbf, cen, w2a, w2c, misc):
    be = 8000
    grid = eadd.shape[0] // be
    whole = lambda s: pl.BlockSpec(s, lambda i: tuple(0 for _ in s))
    return pl.pallas_call(
        _edge_body,
        grid=(grid,),
        in_specs=[
            pl.BlockSpec((be, TW), lambda i: (i, 0)),
            whole((NUM_RBF, 256)),
            whole((1, NUM_RBF)),
            whole((1, HIDDEN)),
            whole((1, HIDDEN)),
            whole((1, HIDDEN)),
        ],
        out_specs=pl.BlockSpec((be, 32), lambda i: (i, 0)),
        out_shape=jax.ShapeDtypeStruct((eadd.shape[0], 32), jnp.float32),
    )(eadd, wrbf, cen, w2a, w2c, misc)


SUP = 10
NSUP = NCHUNK // SUP


def _scatter_body(h_hbm, row2_hbm, col2_hbm, wm_hbm, z_hbm, out_hbm,
                  idxr2, idxc2, hbuf0, hbuf1, wf0, wf1, msg0, msg1, acc,
                  semh0, semh1, semw0, semw1, sems0, sems1):
    c = lax.axis_index("c")
    s = lax.axis_index("s")

    @pl.when(s == 0)
    def _init():
        pltpu.sync_copy(z_hbm, acc)

    plsc.subcore_barrier()

    hbufs = (hbuf0, hbuf1)
    wfs = (wf0, wf1)
    msgs = (msg0, msg1)
    semhs = (semh0, semh1)
    semws = (semw0, semw1)
    semss = (sems0, sems1)

    @pl.when(c == 0)
    def _msg_flow():
        def super_body(u, carry):
            cr = s * NCHUNK + u * SUP
            pltpu.sync_copy(row2_hbm.at[s, u], idxr2)
            pltpu.sync_copy(col2_hbm.at[s, u], idxc2)
            ch = [None, None]
            cw = [None, None]
            sa = [None, None]
            ch[0] = pltpu.async_copy(h_hbm.at[idxc2.at[0]], hbuf0, semh0)
            cw[0] = pltpu.async_copy(
                wm_hbm.at[pl.ds(cr * K * 32, K * 32)], wf0, semw0)
            for j in range(SUP):
                pj = j % 2
                if j + 1 < SUP:
                    ch[1 - pj] = pltpu.async_copy(
                        h_hbm.at[idxc2.at[j + 1]], hbufs[1 - pj], semhs[1 - pj])
                    cw[1 - pj] = pltpu.async_copy(
                        wm_hbm.at[pl.ds((cr + j + 1) * K * 32, K * 32)],
                        wfs[1 - pj], semws[1 - pj])
                ch[pj].wait()
                cw[pj].wait()
                if j >= 2:
                    sa[pj].wait()
                mb = msgs[pj]
                hb = hbufs[pj]
                wb = wfs[pj]

                def edge(e, ecarry, _mb=mb, _hb=hb, _wb=wb):
                    wv = _wb[pl.ds(e * 32, L)]
                    for jj in range(HIDDEN // L):
                        _mb[e, pl.ds(jj * L, L)] = wv * _hb[e, pl.ds(jj * L, L)]
                    return ecarry

                lax.fori_loop(0, K, edge, 0)
                sa[pj] = pltpu.async_copy(mb, acc.at[idxr2.at[j]], semss[pj],
                                          add=True)
            sa[0].wait()
            sa[1].wait()
            return carry

        lax.fori_loop(0, NSUP, super_body, 0)

    @pl.when(c == 1)
    def _aux_flow():
        zv = jnp.zeros((L,), jnp.float32)

        def zrow(e, carry):
            for j in range(1, HIDDEN // L):
                msg0[e, pl.ds(j * L, L)] = zv
                msg1[e, pl.ds(j * L, L)] = zv
            return carry

        lax.fori_loop(0, K, zrow, 0)

        def super_body(u, carry):
            cr = s * NCHUNK + u * SUP
            pltpu.sync_copy(row2_hbm.at[s, u], idxr2)
            cw = [None, None]
            sa = [None, None]
            cw[0] = pltpu.async_copy(
                wm_hbm.at[pl.ds(cr * K * 32, K * 32)], wf0, semw0)
            for j in range(SUP):
                pj = j % 2
                if j + 1 < SUP:
                    cw[1 - pj] = pltpu.async_copy(
                        wm_hbm.at[pl.ds((cr + j + 1) * K * 32, K * 32)],
                        wfs[1 - pj], semws[1 - pj])
                cw[pj].wait()
                if j >= 2:
                    sa[pj].wait()
                mb = msgs[pj]
                wb = wfs[pj]

                def edge(e, ecarry, _mb=mb, _wb=wb):
                    _mb[e, pl.ds(0, L)] = _wb[pl.ds(e * 32 + L, L)]
                    return ecarry

                lax.fori_loop(0, K, edge, 0)
                sa[pj] = pltpu.async_copy(mb, acc.at[idxr2.at[j]], semss[pj],
                                          add=True)
            sa[0].wait()
            sa[1].wait()
            return carry

        lax.fori_loop(0, NSUP, super_body, 0)

    plsc.subcore_barrier()

    @pl.when(s == 0)
    def _drain():
        pltpu.sync_copy(acc, out_hbm.at[c])


def _scatter(h, row2, col2, wm, zeros_acc):
    mesh = plsc.VectorSubcoreMesh(core_axis_name="c", subcore_axis_name="s")
    f = pl.kernel(
        _scatter_body,
        out_type=jax.ShapeDtypeStruct((NC, N_NODES, HIDDEN), jnp.float32),
        mesh=mesh,
        scratch_types=[
            pltpu.VMEM((SUP, K), jnp.int32),
            pltpu.VMEM((SUP, K), jnp.int32),
            pltpu.VMEM((K, HIDDEN), jnp.float32),
            pltpu.VMEM((K, HIDDEN), jnp.float32),
            pltpu.VMEM((K * 32,), jnp.float32),
            pltpu.VMEM((K * 32,), jnp.float32),
            pltpu.VMEM((K, HIDDEN), jnp.float32),
            pltpu.VMEM((K, HIDDEN), jnp.float32),
            pltpu.VMEM_SHARED((N_NODES, HIDDEN), jnp.float32),
            pltpu.SemaphoreType.DMA,
            pltpu.SemaphoreType.DMA,
            pltpu.SemaphoreType.DMA,
            pltpu.SemaphoreType.DMA,
            pltpu.SemaphoreType.DMA,
            pltpu.SemaphoreType.DMA,
        ],
    )
    return f(h, row2, col2, wm, zeros_acc)


def _ln(v, g, b):
    mu = jnp.mean(v, axis=-1, keepdims=True)
    var = jnp.mean((v - mu) ** 2, axis=-1, keepdims=True)
    return (v - mu) / jnp.sqrt(var + 1e-5) * g + b


def _final_body(h_ref, x_ref, acc_ref, mw1h_ref, mw1a_ref, mb1_ref,
                mw2_ref, mb2_ref, fw1_ref, fb1_ref, fw2_ref, fb2_ref,
                g1_ref, b1_ref, g2_ref, b2_ref, xn_ref, h2_ref):
    aux = acc_ref[1]
    cnt = jnp.maximum(aux[:, 3:4], 1.0)
    agg = acc_ref[0] / cnt
    dx = aux[:, 0:3] / cnt
    xn_ref[...] = x_ref[...] + dx
    h = h_ref[...]
    t = jax.nn.relu(
        jnp.dot(h, mw1h_ref[...], preferred_element_type=jnp.float32)
        + jnp.dot(agg, mw1a_ref[...], preferred_element_type=jnp.float32)
        + mb1_ref[...])
    delta_h = jnp.dot(t, mw2_ref[...], preferred_element_type=jnp.float32) + mb2_ref[...]
    h1 = _ln(h + delta_h, g1_ref[...], b1_ref[...])
    h1 = jax.nn.relu(h1)
    f = jax.nn.relu(jnp.dot(h1, fw1_ref[...], preferred_element_type=jnp.float32) + fb1_ref[...])
    ffn = jnp.dot(f, fw2_ref[...], preferred_element_type=jnp.float32) + fb2_ref[...]
    h2_ref[...] = _ln(h1 + ffn, g2_ref[...], b2_ref[...])


def _finalize(h, x, acc, p):
    nb = 5000
    grid = N_NODES // nb
    whole = lambda s: pl.BlockSpec(s, lambda i: tuple(0 for _ in s))
    return pl.pallas_call(
        _final_body,
        grid=(grid,),
        in_specs=[
            pl.BlockSpec((nb, HIDDEN), lambda i: (i, 0)),
            pl.BlockSpec((nb, 3), lambda i: (i, 0)),
            pl.BlockSpec((NC, nb, HIDDEN), lambda i: (0, i, 0)),
            whole((HIDDEN, HIDDEN)),
            whole((HIDDEN, HIDDEN)),
            whole((1, HIDDEN)),
            whole((HIDDEN, HIDDEN)),
            whole((1, HIDDEN)),
            whole((HIDDEN, FF)),
            whole((1, FF)),
            whole((FF, HIDDEN)),
            whole((1, HIDDEN)),
            whole((1, HIDDEN)),
            whole((1, HIDDEN)),
            whole((1, HIDDEN)),
            whole((1, HIDDEN)),
        ],
        out_specs=[
            pl.BlockSpec((nb, 3), lambda i: (i, 0)),
            pl.BlockSpec((nb, HIDDEN), lambda i: (i, 0)),
        ],
        out_shape=[
            jax.ShapeDtypeStruct((N_NODES, 3), jnp.float32),
            jax.ShapeDtypeStruct((N_NODES, HIDDEN), jnp.float32),
        ],
    )(h, x, acc,
      p['msg_w1'][:HIDDEN], p['msg_w1'][HIDDEN:], p['msg_b1'].reshape(1, -1),
      p['msg_w2'], p['msg_b2'].reshape(1, -1),
      p['ffn_w1'], p['ffn_b1'].reshape(1, -1),
      p['ffn_w2'], p['ffn_b2'].reshape(1, -1),
      p['ln1_g'].reshape(1, -1), p['ln1_b'].reshape(1, -1),
      p['ln2_g'].reshape(1, -1), p['ln2_b'].reshape(1, -1))


def kernel(x, h, edge_index, params):
    p = params
    row = edge_index[0]
    col = edge_index[1]
    xp = jnp.pad(x, ((0, 0), (0, HIDDEN - 3)))
    wr = jnp.concatenate([p['att_w1'][:HIDDEN], p['coord_w1'][:HIDDEN]], axis=1)
    wc = jnp.concatenate([p['att_w1'][HIDDEN:2 * HIDDEN],
                          p['coord_w1'][HIDDEN:2 * HIDDEN]], axis=1)
    wrbf = jnp.concatenate([p['att_w1'][2 * HIDDEN:],
                            p['coord_w1'][2 * HIDDEN:]], axis=1)
    cen = jnp.linspace(0.0, CUTOFF, NUM_RBF).reshape(1, -1).astype(jnp.float32)
    bcat = jnp.concatenate([p['att_b1'], p['coord_b1']]).reshape(1, -1)
    w2a = p['att_w2'].reshape(1, -1)
    w2c = p['coord_w2'].reshape(1, -1)
    misc = jnp.zeros((1, HIDDEN), jnp.float32)
    misc = misc.at[0, 0].set(p['att_b2'][0]).at[0, 1].set(p['coord_b2'][0])

    trow, tcol = _precompute(h, wr, wc, xp, bcat)
    eadd = _gather(trow, tcol, row.reshape(NW, GNSUP, GSUP, K),
                   col.reshape(NW, GNSUP, GSUP, K))
    w32 = _edgewise(eadd, wrbf, cen, w2a, w2c, misc)
    zeros_acc = jnp.zeros((N_NODES, HIDDEN), jnp.float32)
    acc = _scatter(h, row.reshape(NS, NSUP, SUP, K), col.reshape(NS, NSUP, SUP, K),
                   w32.reshape(-1), zeros_acc)
    x_new, h2 = _finalize(h, x, acc, p)
    return (x_new, h2)
